# baseline (speedup 1.0000x reference)
import functools

import jax
import jax.numpy as jnp
from jax import lax
from jax.experimental import pallas as pl
from jax.experimental.pallas import tpu as pltpu
from jax.experimental.pallas import tpu_sc as plsc

N_ENT = 100000
N_FACTORS = 64
BATCH = 16384

NC = 2
NS = 16
L = 16
NW = NC * NS
B_PER_W = BATCH // NW
IDX_CHUNK = 128
N_CHUNKS = B_PER_W // IDX_CHUNK
N_GROUPS = B_PER_W // L


def _sc_body(heads_hbm, tails_hbm, emb_h_hbm, emb_t_hbm, out_hbm,
             idx_h, idx_t, rows_h, rows_t, out_v, sem):
    wid = lax.axis_index("s") * NC + lax.axis_index("c")

    pltpu.sync_copy(heads_hbm.at[wid], idx_h)
    pltpu.sync_copy(tails_hbm.at[wid], idx_t)

    copies = []
    for j in range(N_CHUNKS):
        sl = pl.ds(j * IDX_CHUNK, IDX_CHUNK)
        copies.append(pltpu.async_copy(emb_h_hbm.at[idx_h.at[j]],
                                       rows_h.at[sl], sem))
        copies.append(pltpu.async_copy(emb_t_hbm.at[idx_t.at[j]],
                                       rows_t.at[sl], sem))
    for c in copies:
        c.wait()

    iota = lax.broadcasted_iota(jnp.int32, (L,), 0)

    def group(g, _):
        row_ids = g * L + iota
        acc0 = jnp.zeros((L,), jnp.float32)
        acc1 = jnp.zeros((L,), jnp.float32)
        for d in range(N_FACTORS // 2):
            c0 = jnp.full((L,), 2 * d, jnp.int32)
            c1 = jnp.full((L,), 2 * d + 1, jnp.int32)
            h0 = plsc.load_gather(rows_h, [row_ids, c0])
            t0 = plsc.load_gather(rows_t, [row_ids, c0])
            h1 = plsc.load_gather(rows_h, [row_ids, c1])
            t1 = plsc.load_gather(rows_t, [row_ids, c1])
            acc0 = acc0 + h0 * t0
            acc1 = acc1 + h1 * t1
        out_v[pl.ds(g * L, L)] = acc0 + acc1
        return ()

    lax.fori_loop(0, N_GROUPS, group, ())

    pltpu.sync_copy(out_v, out_hbm.at[pl.ds(wid * B_PER_W, B_PER_W)])


@jax.jit
def kernel(heads, tails, emb_heads, emb_tails):
    heads3 = heads.astype(jnp.int32).reshape(NW, N_CHUNKS, IDX_CHUNK)
    tails3 = tails.astype(jnp.int32).reshape(NW, N_CHUNKS, IDX_CHUNK)

    mesh = plsc.VectorSubcoreMesh(core_axis_name="c", subcore_axis_name="s")
    run = pl.kernel(
        _sc_body,
        out_type=jax.ShapeDtypeStruct((BATCH,), jnp.float32),
        mesh=mesh,
        compiler_params=pltpu.CompilerParams(
            needs_layout_passes=False, use_tc_tiling_on_sc=False),
        scratch_types=[
            pltpu.VMEM((N_CHUNKS, IDX_CHUNK), jnp.int32),
            pltpu.VMEM((N_CHUNKS, IDX_CHUNK), jnp.int32),
            pltpu.VMEM((B_PER_W, N_FACTORS), jnp.float32),
            pltpu.VMEM((B_PER_W, N_FACTORS), jnp.float32),
            pltpu.VMEM((B_PER_W,), jnp.float32),
            pltpu.SemaphoreType.DMA,
        ],
    )
    return run(heads3, tails3, emb_heads, emb_tails)


# speedup vs baseline: 1.1653x; 1.1653x over previous
import jax
import jax.numpy as jnp
from jax import lax
from jax.experimental import pallas as pl
from jax.experimental.pallas import tpu as pltpu
from jax.experimental.pallas import tpu_sc as plsc

N_ENT = 100000
N_FACTORS = 64
BATCH = 16384

NC = 2
NS = 16
L = 16
NW = NC * NS
B_PER_W = BATCH // NW
IDX_CHUNK = 128
N_CHUNKS = B_PER_W // IDX_CHUNK
ROW_UNROLL = 8


def _sc_body(heads_hbm, tails_hbm, emb_h_hbm, emb_t_hbm, out_hbm,
             idx_h, idx_t, rows_h, rows_t, out_v, sem0, sem1):
    wid = lax.axis_index("s") * NC + lax.axis_index("c")
    base = wid * B_PER_W

    pltpu.sync_copy(heads_hbm.at[pl.ds(base, B_PER_W)], idx_h)
    pltpu.sync_copy(tails_hbm.at[pl.ds(base, B_PER_W)], idx_t)

    sems = (sem0, sem1)

    def fire(j):
        sl = pl.ds(j * IDX_CHUNK, IDX_CHUNK)
        s = sems[j % 2]
        return (pltpu.async_copy(emb_h_hbm.at[idx_h.at[sl]], rows_h.at[sl], s),
                pltpu.async_copy(emb_t_hbm.at[idx_t.at[sl]], rows_t.at[sl], s))

    iota = lax.broadcasted_iota(jnp.int32, (L,), 0)
    lane0 = iota == 0

    def row_dot(r):
        h0 = rows_h[r, pl.ds(0, L)]
        h1 = rows_h[r, pl.ds(L, L)]
        h2 = rows_h[r, pl.ds(2 * L, L)]
        h3 = rows_h[r, pl.ds(3 * L, L)]
        t0 = rows_t[r, pl.ds(0, L)]
        t1 = rows_t[r, pl.ds(L, L)]
        t2 = rows_t[r, pl.ds(2 * L, L)]
        t3 = rows_t[r, pl.ds(3 * L, L)]
        s = (h0 * t0 + h1 * t1) + (h2 * t2 + h3 * t3)
        tot = jnp.broadcast_to(jnp.sum(s), (L,))
        plsc.store_scatter(out_v, [jnp.full((L,), r, jnp.int32)], tot,
                           mask=lane0)

    inflight = fire(0)
    for j in range(N_CHUNKS):
        nxt = fire(j + 1) if j + 1 < N_CHUNKS else None
        for c in inflight:
            c.wait()
        inflight = nxt

        def block(i, _):
            r0 = j * IDX_CHUNK + i * ROW_UNROLL
            for k in range(ROW_UNROLL):
                row_dot(r0 + k)
            return ()

        lax.fori_loop(0, IDX_CHUNK // ROW_UNROLL, block, ())

    pltpu.sync_copy(out_v, out_hbm.at[pl.ds(base, B_PER_W)])


@jax.jit
def kernel(heads, tails, emb_heads, emb_tails):
    mesh = plsc.VectorSubcoreMesh(core_axis_name="c", subcore_axis_name="s")
    run = pl.kernel(
        _sc_body,
        out_type=jax.ShapeDtypeStruct((BATCH,), jnp.float32),
        mesh=mesh,
        compiler_params=pltpu.CompilerParams(
            needs_layout_passes=False, use_tc_tiling_on_sc=False),
        scratch_types=[
            pltpu.VMEM((B_PER_W,), jnp.int32),
            pltpu.VMEM((B_PER_W,), jnp.int32),
            pltpu.VMEM((B_PER_W, N_FACTORS), jnp.float32),
            pltpu.VMEM((B_PER_W, N_FACTORS), jnp.float32),
            pltpu.VMEM((B_PER_W,), jnp.float32),
            pltpu.SemaphoreType.DMA,
            pltpu.SemaphoreType.DMA,
        ],
    )
    return run(heads.astype(jnp.int32), tails.astype(jnp.int32),
               emb_heads, emb_tails)
